# baseline (speedup 1.0000x reference)
import functools

import jax
import jax.numpy as jnp
from jax import lax
from jax.experimental import pallas as pl
from jax.experimental.pallas import tpu as pltpu
from jax.experimental.pallas import tpu_sc as plsc

B = 16384
D = 16

NC = 2
NS = 16
NW = NC * NS
CH = 128
GPW = B // (NW * CH)

BLK = 2048


def _sc_gather(uid_table, mid_table, uid_idx, mid_idx):
  mesh = plsc.VectorSubcoreMesh(core_axis_name="c", subcore_axis_name="s")

  @functools.partial(
      pl.kernel,
      out_type=(
          jax.ShapeDtypeStruct((NW, GPW, CH, D), jnp.float32),
          jax.ShapeDtypeStruct((NW, GPW, CH, D), jnp.float32),
      ),
      mesh=mesh,
      scratch_types=[
          pltpu.VMEM((GPW, CH), jnp.int32),
          pltpu.VMEM((GPW, CH), jnp.int32),
          pltpu.VMEM((GPW, CH, D), jnp.float32),
          pltpu.VMEM((GPW, CH, D), jnp.float32),
          pltpu.SemaphoreType.DMA,
      ],
      compiler_params=pltpu.CompilerParams(use_tc_tiling_on_sc=False),
  )
  def body(uid_t, mid_t, uidx, midx, e_uid, e_mid,
           uidx_v, midx_v, urows, mrows, sem):
    wid = lax.axis_index("s") * NC + lax.axis_index("c")
    pltpu.sync_copy(uidx.at[wid], uidx_v)
    pltpu.sync_copy(midx.at[wid], midx_v)
    copies = []
    for j in range(GPW):
      copies.append(pltpu.async_copy(uid_t.at[uidx_v.at[j]], urows.at[j], sem))
      copies.append(pltpu.async_copy(mid_t.at[midx_v.at[j]], mrows.at[j], sem))
    for c in copies:
      c.wait()
    pltpu.sync_copy(urows, e_uid.at[wid])
    pltpu.sync_copy(mrows, e_mid.at[wid])

  return body(uid_table, mid_table, uid_idx, mid_idx)


def _tc_math(e_uid, e_mid, gender2, age2, job2, gender_table, age_table,
             job_table, W1, b1r, W2r, b2r, Wsu, Wsm, bsr):
  def body(eu_ref, em_ref, g_ref, a_ref, j_ref, gt_ref, at_ref, jt_ref,
           w1_ref, b1_ref, w2_ref, b2_ref, wsu_ref, wsm_ref, bs_ref,
           out_ref, lam_ref):
    e_mid = em_ref[...]
    w1_top = w1_ref[0:D, :]
    w1_bot = w1_ref[D:2 * D, :]
    m1 = jnp.dot(e_mid, w1_top, preferred_element_type=jnp.float32) + b1_ref[...]

    def att_branch(idx_ref, tab_ref, vocab):
      idx = idx_ref[...]
      iot = lax.broadcasted_iota(jnp.int32, (BLK, vocab), 1)
      oh = (idx == iot).astype(jnp.float32)
      e = jnp.dot(oh, tab_ref[...], preferred_element_type=jnp.float32)
      h = jnp.tanh(m1 + jnp.dot(e, w1_bot, preferred_element_type=jnp.float32))
      s = jnp.sum(h * w2_ref[...], axis=1, keepdims=True) + b2_ref[...]
      return e, s

    e_g, s_g = att_branch(g_ref, gt_ref, 2)
    e_a, s_a = att_branch(a_ref, at_ref, 7)
    e_j, s_j = att_branch(j_ref, jt_ref, 21)

    s = jnp.concatenate([s_g, s_a, s_j], axis=1)
    m = jnp.max(s, axis=1, keepdims=True)
    ex = jnp.exp(s - m)
    lam = ex / jnp.sum(ex, axis=1, keepdims=True)
    lam_ref[...] = lam

    fu = (lam[:, 0:1] * e_g + lam[:, 1:2] * e_a + lam[:, 2:3] * e_j
          + eu_ref[...])
    out_ref[...] = (jnp.sum(fu * wsu_ref[...], axis=1, keepdims=True)
                    + jnp.sum(e_mid * wsm_ref[...], axis=1, keepdims=True)
                    + bs_ref[...])

  nblk = B // BLK
  full = lambda shape: pl.BlockSpec(shape, lambda i: tuple(0 for _ in shape))
  blk = lambda shape: pl.BlockSpec(shape, lambda i: (i,) + (0,) * (len(shape) - 1))
  return pl.pallas_call(
      body,
      grid=(nblk,),
      in_specs=[
          blk((BLK, D)), blk((BLK, D)),
          blk((BLK, 1)), blk((BLK, 1)), blk((BLK, 1)),
          full((2, D)), full((7, D)), full((21, D)),
          full((2 * D, D)), full((1, D)), full((1, D)), full((1, 1)),
          full((1, D)), full((1, D)), full((1, 1)),
      ],
      out_specs=[blk((BLK, 1)), blk((BLK, 3))],
      out_shape=[
          jax.ShapeDtypeStruct((B, 1), jnp.float32),
          jax.ShapeDtypeStruct((B, 3), jnp.float32),
      ],
  )(e_uid, e_mid, gender2, age2, job2, gender_table, age_table, job_table,
    W1, b1r, W2r, b2r, Wsu, Wsm, bsr)


def kernel(uid_table, gender_table, age_table, job_table, mid_table,
           W1, b1, W2, b2, W_svd, b_svd,
           uid, gender, age, job, mid):
  uid_idx = uid.astype(jnp.int32).reshape(NW, GPW, CH)
  mid_idx = mid.astype(jnp.int32).reshape(NW, GPW, CH)
  e_uid, e_mid = _sc_gather(uid_table, mid_table, uid_idx, mid_idx)
  e_uid = e_uid.reshape(B, D)
  e_mid = e_mid.reshape(B, D)

  out, lam = _tc_math(
      e_uid, e_mid,
      gender.astype(jnp.int32).reshape(B, 1),
      age.astype(jnp.int32).reshape(B, 1),
      job.astype(jnp.int32).reshape(B, 1),
      gender_table, age_table, job_table,
      W1, b1.reshape(1, D), W2.reshape(1, D), b2.reshape(1, 1),
      W_svd[:D].reshape(1, D), W_svd[D:].reshape(1, D), b_svd.reshape(1, 1))
  return (out, lam.reshape(B, 3, 1))
